# Initial kernel scaffold; baseline (speedup 1.0000x reference)
import jax
import jax.numpy as jnp
from jax.experimental import pallas as pl


def kernel(z, y_gene, edges, running_mean, running_var, smoothed_mean, smoothed_var):
    raise NotImplementedError("write your pallas kernel here")


# speedup vs baseline: 45.5295x; 45.5295x over previous
import jax
import jax.numpy as jnp
from jax.experimental import pallas as pl

_EPS = 1e-6
_NBINS = 100
_D = 128
_BP = 128
_R = 1024


def _prep_body(rm_ref, rv_ref, sm_ref, sv_ref, hi_ref, lo_ref):
    scale = jnp.sqrt(sv_ref[...] + _EPS) / jnp.sqrt(rv_ref[...] + _EPS)
    offset = sm_ref[...] - rm_ref[...] * scale
    comb = jnp.concatenate([scale, offset], axis=1)
    hi = comb.astype(jnp.bfloat16)
    lo = (comb - hi.astype(jnp.float32)).astype(jnp.bfloat16)
    hi_ref[...] = hi
    lo_ref[...] = lo


def _main_body(y_ref, edges_ref, hi_ref, lo_ref, z_ref, out_ref):
    y = y_ref[...]
    edges = edges_ref[...]
    cnt = jnp.sum((y >= edges).astype(jnp.int32), axis=1, keepdims=True)
    idx = jnp.clip(cnt - 1, 0, _NBINS - 1)
    lanes = jax.lax.broadcasted_iota(jnp.int32, (_R, _BP), 1)
    oh = (lanes == idx).astype(jnp.bfloat16)
    so = jnp.dot(oh, hi_ref[...], preferred_element_type=jnp.float32)
    so = so + jnp.dot(oh, lo_ref[...], preferred_element_type=jnp.float32)
    out_ref[...] = z_ref[...] * so[:, :_D] + so[:, _D:]


def kernel(z, y_gene, edges, running_mean, running_var, smoothed_mean, smoothed_var):
    n, d = z.shape
    pad = ((0, _BP - _NBINS), (0, 0))
    rm = jnp.pad(running_mean, pad)
    rv = jnp.pad(running_var, pad, constant_values=1.0)
    sm = jnp.pad(smoothed_mean, pad)
    sv = jnp.pad(smoothed_var, pad, constant_values=1.0)
    edges_pad = jnp.full((1, _BP), jnp.inf, dtype=jnp.float32)
    edges_pad = edges_pad.at[0, : _NBINS + 1].set(edges)

    hi, lo = pl.pallas_call(
        _prep_body,
        out_shape=[
            jax.ShapeDtypeStruct((_BP, 2 * _D), jnp.bfloat16),
            jax.ShapeDtypeStruct((_BP, 2 * _D), jnp.bfloat16),
        ],
    )(rm, rv, sm, sv)

    y2d = y_gene.reshape(n, 1)
    out = pl.pallas_call(
        _main_body,
        grid=(n // _R,),
        in_specs=[
            pl.BlockSpec((_R, 1), lambda i: (i, 0)),
            pl.BlockSpec((1, _BP), lambda i: (0, 0)),
            pl.BlockSpec((_BP, 2 * _D), lambda i: (0, 0)),
            pl.BlockSpec((_BP, 2 * _D), lambda i: (0, 0)),
            pl.BlockSpec((_R, _D), lambda i: (i, 0)),
        ],
        out_specs=pl.BlockSpec((_R, _D), lambda i: (i, 0)),
        out_shape=jax.ShapeDtypeStruct((n, d), jnp.float32),
    )(y2d, edges_pad, hi, lo, z)
    return out


# speedup vs baseline: 72.3905x; 1.5900x over previous
import jax
import jax.numpy as jnp
from jax.experimental import pallas as pl

_EPS = 1e-6
_NBINS = 100
_D = 128
_BP = 128
_R = 2048


def _prep_body(rm_ref, rv_ref, sm_ref, sv_ref, hi_ref, lo_ref):
    scale = jnp.sqrt(sv_ref[...] + _EPS) / jnp.sqrt(rv_ref[...] + _EPS)
    offset = sm_ref[...] - rm_ref[...] * scale
    comb = jnp.concatenate([scale, offset], axis=1)
    hi = comb.astype(jnp.bfloat16)
    lo = (comb - hi.astype(jnp.float32)).astype(jnp.bfloat16)
    hi_ref[...] = hi
    lo_ref[...] = lo


def _main_body(y_ref, edges_ref, hi_ref, lo_ref, z_ref, out_ref):
    y = y_ref[...]
    e = edges_ref[...]
    oh = jnp.logical_and(y >= e[0:1, :], y < e[1:2, :]).astype(jnp.bfloat16)
    so = jnp.dot(oh, hi_ref[...], preferred_element_type=jnp.float32)
    so = so + jnp.dot(oh, lo_ref[...], preferred_element_type=jnp.float32)
    out_ref[...] = z_ref[...] * so[:, :_D] + so[:, _D:]


def kernel(z, y_gene, edges, running_mean, running_var, smoothed_mean, smoothed_var):
    n, d = z.shape
    pad = ((0, _BP - _NBINS), (0, 0))
    rm = jnp.pad(running_mean, pad)
    rv = jnp.pad(running_var, pad, constant_values=1.0)
    sm = jnp.pad(smoothed_mean, pad)
    sv = jnp.pad(smoothed_var, pad, constant_values=1.0)
    edges_pad = jnp.full((2, _BP), jnp.inf, dtype=jnp.float32)
    edges_pad = edges_pad.at[0, :_NBINS].set(edges[:_NBINS])
    edges_pad = edges_pad.at[1, :_NBINS].set(edges[1 : _NBINS + 1])

    hi, lo = pl.pallas_call(
        _prep_body,
        out_shape=[
            jax.ShapeDtypeStruct((_BP, 2 * _D), jnp.bfloat16),
            jax.ShapeDtypeStruct((_BP, 2 * _D), jnp.bfloat16),
        ],
    )(rm, rv, sm, sv)

    y2d = y_gene.reshape(n, 1)
    out = pl.pallas_call(
        _main_body,
        grid=(n // _R,),
        in_specs=[
            pl.BlockSpec((_R, 1), lambda i: (i, 0)),
            pl.BlockSpec((2, _BP), lambda i: (0, 0)),
            pl.BlockSpec((_BP, 2 * _D), lambda i: (0, 0)),
            pl.BlockSpec((_BP, 2 * _D), lambda i: (0, 0)),
            pl.BlockSpec((_R, _D), lambda i: (i, 0)),
        ],
        out_specs=pl.BlockSpec((_R, _D), lambda i: (i, 0)),
        out_shape=jax.ShapeDtypeStruct((n, d), jnp.float32),
    )(y2d, edges_pad, hi, lo, z)
    return out


# speedup vs baseline: 98.1766x; 1.3562x over previous
import jax
import jax.numpy as jnp
from jax.experimental import pallas as pl

_EPS = 1e-6
_NBINS = 100
_D = 128
_BP = 128
_R = 2048


def _prep_body(rm_ref, rv_ref, sm_ref, sv_ref, hi_ref):
    scale = jnp.sqrt(sv_ref[...] + _EPS) / jnp.sqrt(rv_ref[...] + _EPS)
    offset = sm_ref[...] - rm_ref[...] * scale
    comb = jnp.concatenate([scale, offset], axis=1)
    hi = comb.astype(jnp.bfloat16)
    lo = (comb - hi.astype(jnp.float32)).astype(jnp.bfloat16)
    hi_ref[...] = jnp.concatenate([hi, lo], axis=0)


def _main_body(y_ref, edges_ref, hilo_ref, z_ref, out_ref):
    y16 = y_ref[...]
    e = edges_ref[...]
    slabs = []
    for g in range(_R // _D):
        yc = jnp.transpose(y16[g : g + 1, :])
        slabs.append(
            jnp.logical_and(yc >= e[0:1, :], yc < e[1:2, :]).astype(jnp.bfloat16)
        )
    oh = jnp.concatenate(slabs, axis=0)
    oh2 = jnp.concatenate([oh, oh], axis=1)
    so = jnp.dot(oh2, hilo_ref[...], preferred_element_type=jnp.float32)
    out_ref[...] = z_ref[...] * so[:, :_D] + so[:, _D:]


def kernel(z, y_gene, edges, running_mean, running_var, smoothed_mean, smoothed_var):
    n, d = z.shape
    pad = ((0, _BP - _NBINS), (0, 0))
    rm = jnp.pad(running_mean, pad)
    rv = jnp.pad(running_var, pad, constant_values=1.0)
    sm = jnp.pad(smoothed_mean, pad)
    sv = jnp.pad(smoothed_var, pad, constant_values=1.0)
    edges_pad = jnp.full((2, _BP), jnp.inf, dtype=jnp.float32)
    edges_pad = edges_pad.at[0, :_NBINS].set(edges[:_NBINS])
    edges_pad = edges_pad.at[1, :_NBINS].set(edges[1 : _NBINS + 1])

    hilo = pl.pallas_call(
        _prep_body,
        out_shape=jax.ShapeDtypeStruct((2 * _BP, 2 * _D), jnp.bfloat16),
    )(rm, rv, sm, sv)

    y2d = y_gene.reshape(n // _D, _D)
    out = pl.pallas_call(
        _main_body,
        grid=(n // _R,),
        in_specs=[
            pl.BlockSpec((_R // _D, _D), lambda i: (i, 0)),
            pl.BlockSpec((2, _BP), lambda i: (0, 0)),
            pl.BlockSpec((2 * _BP, 2 * _D), lambda i: (0, 0)),
            pl.BlockSpec((_R, _D), lambda i: (i, 0)),
        ],
        out_specs=pl.BlockSpec((_R, _D), lambda i: (i, 0)),
        out_shape=jax.ShapeDtypeStruct((n, d), jnp.float32),
    )(y2d, edges_pad, hilo, z)
    return out


# speedup vs baseline: 131.5802x; 1.3402x over previous
import jax
import jax.numpy as jnp
from jax.experimental import pallas as pl

_EPS = 1e-6
_NBINS = 100
_D = 128
_BP = 128
_R = 4096


def _prep_body(rm_ref, rv_ref, sm_ref, sv_ref, hi_ref):
    scale = jnp.sqrt(sv_ref[...] + _EPS) / jnp.sqrt(rv_ref[...] + _EPS)
    offset = sm_ref[...] - rm_ref[...] * scale
    comb = jnp.concatenate([scale, offset], axis=1)
    hi = comb.astype(jnp.bfloat16)
    lo = (comb - hi.astype(jnp.float32)).astype(jnp.bfloat16)
    hi_ref[...] = jnp.concatenate([hi, lo], axis=0)


def _main_body(y_ref, edges_ref, hilo_ref, z_ref, out_ref):
    y16 = y_ref[...]
    e = edges_ref[...]
    slabs = []
    for g in range(_R // _D):
        yc = jnp.transpose(y16[g : g + 1, :])
        slabs.append(
            jnp.logical_and(yc >= e[0:1, :], yc < e[1:2, :]).astype(jnp.bfloat16)
        )
    oh = jnp.concatenate(slabs, axis=0)
    oh2 = jnp.concatenate([oh, oh], axis=1)
    so = jnp.dot(oh2, hilo_ref[...], preferred_element_type=jnp.float32)
    out_ref[...] = z_ref[...] * so[:, :_D] + so[:, _D:]


def kernel(z, y_gene, edges, running_mean, running_var, smoothed_mean, smoothed_var):
    n, d = z.shape
    pad = ((0, _BP - _NBINS), (0, 0))
    rm = jnp.pad(running_mean, pad)
    rv = jnp.pad(running_var, pad, constant_values=1.0)
    sm = jnp.pad(smoothed_mean, pad)
    sv = jnp.pad(smoothed_var, pad, constant_values=1.0)
    edges_pad = jnp.full((2, _BP), jnp.inf, dtype=jnp.float32)
    edges_pad = edges_pad.at[0, :_NBINS].set(edges[:_NBINS])
    edges_pad = edges_pad.at[1, :_NBINS].set(edges[1 : _NBINS + 1])

    hilo = pl.pallas_call(
        _prep_body,
        out_shape=jax.ShapeDtypeStruct((2 * _BP, 2 * _D), jnp.bfloat16),
    )(rm, rv, sm, sv)

    y2d = y_gene.reshape(n // _D, _D)
    out = pl.pallas_call(
        _main_body,
        grid=(n // _R,),
        in_specs=[
            pl.BlockSpec((_R // _D, _D), lambda i: (i, 0)),
            pl.BlockSpec((2, _BP), lambda i: (0, 0)),
            pl.BlockSpec((2 * _BP, 2 * _D), lambda i: (0, 0)),
            pl.BlockSpec((_R, _D), lambda i: (i, 0)),
        ],
        out_specs=pl.BlockSpec((_R, _D), lambda i: (i, 0)),
        out_shape=jax.ShapeDtypeStruct((n, d), jnp.float32),
    )(y2d, edges_pad, hilo, z)
    return out


# speedup vs baseline: 155.7501x; 1.1837x over previous
import jax
import jax.numpy as jnp
from jax.experimental import pallas as pl

_EPS = 1e-6
_NBINS = 100
_D = 128
_BP = 128
_R = 8192


def _prep_body(rm_ref, rv_ref, sm_ref, sv_ref, hi_ref):
    scale = jnp.sqrt(sv_ref[...] + _EPS) / jnp.sqrt(rv_ref[...] + _EPS)
    offset = sm_ref[...] - rm_ref[...] * scale
    comb = jnp.concatenate([scale, offset], axis=1)
    hi = comb.astype(jnp.bfloat16)
    lo = (comb - hi.astype(jnp.float32)).astype(jnp.bfloat16)
    hi_ref[...] = jnp.concatenate([hi, lo], axis=0)


def _main_body(y_ref, edges_ref, hilo_ref, z_ref, out_ref):
    y16 = y_ref[...]
    e = edges_ref[...]
    slabs = []
    for g in range(_R // _D):
        yc = jnp.transpose(y16[g : g + 1, :])
        slabs.append(
            jnp.logical_and(yc >= e[0:1, :], yc < e[1:2, :]).astype(jnp.bfloat16)
        )
    oh = jnp.concatenate(slabs, axis=0)
    oh2 = jnp.concatenate([oh, oh], axis=1)
    so = jnp.dot(oh2, hilo_ref[...], preferred_element_type=jnp.float32)
    out_ref[...] = z_ref[...] * so[:, :_D] + so[:, _D:]


def kernel(z, y_gene, edges, running_mean, running_var, smoothed_mean, smoothed_var):
    n, d = z.shape
    pad = ((0, _BP - _NBINS), (0, 0))
    rm = jnp.pad(running_mean, pad)
    rv = jnp.pad(running_var, pad, constant_values=1.0)
    sm = jnp.pad(smoothed_mean, pad)
    sv = jnp.pad(smoothed_var, pad, constant_values=1.0)
    edges_pad = jnp.full((2, _BP), jnp.inf, dtype=jnp.float32)
    edges_pad = edges_pad.at[0, :_NBINS].set(edges[:_NBINS])
    edges_pad = edges_pad.at[1, :_NBINS].set(edges[1 : _NBINS + 1])

    hilo = pl.pallas_call(
        _prep_body,
        out_shape=jax.ShapeDtypeStruct((2 * _BP, 2 * _D), jnp.bfloat16),
    )(rm, rv, sm, sv)

    y2d = y_gene.reshape(n // _D, _D)
    out = pl.pallas_call(
        _main_body,
        grid=(n // _R,),
        in_specs=[
            pl.BlockSpec((_R // _D, _D), lambda i: (i, 0)),
            pl.BlockSpec((2, _BP), lambda i: (0, 0)),
            pl.BlockSpec((2 * _BP, 2 * _D), lambda i: (0, 0)),
            pl.BlockSpec((_R, _D), lambda i: (i, 0)),
        ],
        out_specs=pl.BlockSpec((_R, _D), lambda i: (i, 0)),
        out_shape=jax.ShapeDtypeStruct((n, d), jnp.float32),
    )(y2d, edges_pad, hilo, z)
    return out


# speedup vs baseline: 172.7383x; 1.1091x over previous
import jax
import jax.numpy as jnp
from jax.experimental import pallas as pl

_EPS = 1e-6
_NBINS = 100
_D = 128
_BP = 128
_R = 16384


def _prep_body(rm_ref, rv_ref, sm_ref, sv_ref, hi_ref):
    scale = jnp.sqrt(sv_ref[...] + _EPS) / jnp.sqrt(rv_ref[...] + _EPS)
    offset = sm_ref[...] - rm_ref[...] * scale
    comb = jnp.concatenate([scale, offset], axis=1)
    hi = comb.astype(jnp.bfloat16)
    lo = (comb - hi.astype(jnp.float32)).astype(jnp.bfloat16)
    hi_ref[...] = jnp.concatenate([hi, lo], axis=0)


def _main_body(y_ref, edges_ref, hilo_ref, z_ref, out_ref):
    y16 = y_ref[...]
    e = edges_ref[...]
    slabs = []
    for g in range(_R // _D):
        yc = jnp.transpose(y16[g : g + 1, :])
        slabs.append(
            jnp.logical_and(yc >= e[0:1, :], yc < e[1:2, :]).astype(jnp.bfloat16)
        )
    oh = jnp.concatenate(slabs, axis=0)
    oh2 = jnp.concatenate([oh, oh], axis=1)
    so = jnp.dot(oh2, hilo_ref[...], preferred_element_type=jnp.float32)
    out_ref[...] = z_ref[...] * so[:, :_D] + so[:, _D:]


def kernel(z, y_gene, edges, running_mean, running_var, smoothed_mean, smoothed_var):
    n, d = z.shape
    pad = ((0, _BP - _NBINS), (0, 0))
    rm = jnp.pad(running_mean, pad)
    rv = jnp.pad(running_var, pad, constant_values=1.0)
    sm = jnp.pad(smoothed_mean, pad)
    sv = jnp.pad(smoothed_var, pad, constant_values=1.0)
    edges_pad = jnp.full((2, _BP), jnp.inf, dtype=jnp.float32)
    edges_pad = edges_pad.at[0, :_NBINS].set(edges[:_NBINS])
    edges_pad = edges_pad.at[1, :_NBINS].set(edges[1 : _NBINS + 1])

    hilo = pl.pallas_call(
        _prep_body,
        out_shape=jax.ShapeDtypeStruct((2 * _BP, 2 * _D), jnp.bfloat16),
    )(rm, rv, sm, sv)

    y2d = y_gene.reshape(n // _D, _D)
    out = pl.pallas_call(
        _main_body,
        grid=(n // _R,),
        in_specs=[
            pl.BlockSpec((_R // _D, _D), lambda i: (i, 0)),
            pl.BlockSpec((2, _BP), lambda i: (0, 0)),
            pl.BlockSpec((2 * _BP, 2 * _D), lambda i: (0, 0)),
            pl.BlockSpec((_R, _D), lambda i: (i, 0)),
        ],
        out_specs=pl.BlockSpec((_R, _D), lambda i: (i, 0)),
        out_shape=jax.ShapeDtypeStruct((n, d), jnp.float32),
    )(y2d, edges_pad, hilo, z)
    return out
